# Initial kernel scaffold; baseline (speedup 1.0000x reference)
import jax
import jax.numpy as jnp
from jax.experimental import pallas as pl


def kernel(x, mask, edge_index, edge_attr, Wm1, bm1, Wm2, bm2, Wa1, ba1, Wa2, ba2, Wt, bt, Wb1, bb1, Wb2, bb2):
    raise NotImplementedError("write your pallas kernel here")


# speedup vs baseline: 6.8149x; 6.8149x over previous
import functools

import jax
import jax.numpy as jnp
from jax import lax
from jax.experimental import pallas as pl
from jax.experimental.pallas import tpu as pltpu
from jax.experimental.pallas import tpu_sc as plsc

NC = 2
NS = 16
LANES = 16


def _dot(a, b):
    return jnp.dot(a, b, preferred_element_type=jnp.float32)


def _tc_prep(x, mask, Wm1, bm1, Wm2, bm2, Wc, Wr):
    N, F = x.shape
    H = Wc.shape[1]
    bN = 1000

    def body(x_ref, m_ref, wm1, b1, wm2, b2, wc, wr, out_ref):
        t = jnp.maximum(_dot(m_ref[...], wm1[...]) + b1[...], 0.0)
        xp = _dot(t, wm2[...]) + b2[...] + x_ref[...]
        out_ref[:, :H] = _dot(xp, wc[...])
        out_ref[:, H:] = _dot(xp, wr[...])

    return pl.pallas_call(
        body,
        grid=(N // bN,),
        in_specs=[
            pl.BlockSpec((bN, F), lambda i: (i, 0)),
            pl.BlockSpec((bN, F), lambda i: (i, 0)),
            pl.BlockSpec(Wm1.shape, lambda i: (0, 0)),
            pl.BlockSpec((1, bm1.shape[-1]), lambda i: (0, 0)),
            pl.BlockSpec(Wm2.shape, lambda i: (0, 0)),
            pl.BlockSpec((1, bm2.shape[-1]), lambda i: (0, 0)),
            pl.BlockSpec(Wc.shape, lambda i: (0, 0)),
            pl.BlockSpec(Wr.shape, lambda i: (0, 0)),
        ],
        out_specs=pl.BlockSpec((bN, 2 * H), lambda i: (i, 0)),
        out_shape=jax.ShapeDtypeStruct((N, 2 * H), jnp.float32),
    )(x, mask, Wm1, bm1.reshape(1, -1), Wm2, bm2.reshape(1, -1), Wc, Wr)


def _tc_edgepre(ea, W, b):
    E, FE = ea.shape
    H = W.shape[1]
    bE = 8000

    def body(ea_ref, w_ref, b_ref, out_ref):
        out_ref[...] = _dot(ea_ref[...], w_ref[...]) + b_ref[...]

    return pl.pallas_call(
        body,
        grid=(E // bE,),
        in_specs=[
            pl.BlockSpec((bE, FE), lambda i: (i, 0)),
            pl.BlockSpec((FE, H), lambda i: (0, 0)),
            pl.BlockSpec((1, H), lambda i: (0, 0)),
        ],
        out_specs=pl.BlockSpec((bE, H), lambda i: (i, 0)),
        out_shape=jax.ShapeDtypeStruct((E, H), jnp.float32),
    )(ea, W, b.reshape(1, -1))


def _tc_combine_a(accA, degacc, Wa2, ba2):
    _, N, H = accA.shape
    bN = 1000

    def body(acc_ref, dacc_ref, w_ref, b_ref, x1_ref, g_ref, deg_ref, dinv_ref):
        S = acc_ref[0] + acc_ref[1]
        degc = dacc_ref[0][:, 0:1] + dacc_ref[1][:, 0:1]
        x1 = jnp.maximum(_dot(S, w_ref[...]) + degc * b_ref[...], 0.0)
        dinvc = jnp.where(degc > 0, lax.rsqrt(jnp.maximum(degc, 1e-30)), 0.0)
        x1_ref[...] = x1
        g_ref[...] = dinvc * x1
        deg_ref[...] = jnp.broadcast_to(degc, (degc.shape[0], LANES))
        dinv_ref[...] = jnp.broadcast_to(dinvc, (dinvc.shape[0], LANES))

    return pl.pallas_call(
        body,
        grid=(N // bN,),
        in_specs=[
            pl.BlockSpec((2, bN, H), lambda i: (0, i, 0)),
            pl.BlockSpec((2, bN, H), lambda i: (0, i, 0)),
            pl.BlockSpec((H, H), lambda i: (0, 0)),
            pl.BlockSpec((1, H), lambda i: (0, 0)),
        ],
        out_specs=[
            pl.BlockSpec((bN, H), lambda i: (i, 0)),
            pl.BlockSpec((bN, H), lambda i: (i, 0)),
            pl.BlockSpec((bN, LANES), lambda i: (i, 0)),
            pl.BlockSpec((bN, LANES), lambda i: (i, 0)),
        ],
        out_shape=[
            jax.ShapeDtypeStruct((N, H), jnp.float32),
            jax.ShapeDtypeStruct((N, H), jnp.float32),
            jax.ShapeDtypeStruct((N, LANES), jnp.float32),
            jax.ShapeDtypeStruct((N, LANES), jnp.float32),
        ],
    )(accA, degacc, Wa2, ba2.reshape(1, -1))


def _tc_combine_t(accT, dinv16):
    _, N, H = accT.shape
    bN = 1000

    def body(acc_ref, dinv_ref, h_ref, g_ref):
        dinvc = dinv_ref[:, 0:1]
        h = dinvc * (acc_ref[0] + acc_ref[1])
        h_ref[...] = h
        g_ref[...] = dinvc * h

    return pl.pallas_call(
        body,
        grid=(N // bN,),
        in_specs=[
            pl.BlockSpec((2, bN, H), lambda i: (0, i, 0)),
            pl.BlockSpec((bN, LANES), lambda i: (i, 0)),
        ],
        out_specs=[
            pl.BlockSpec((bN, H), lambda i: (i, 0)),
            pl.BlockSpec((bN, H), lambda i: (i, 0)),
        ],
        out_shape=[
            jax.ShapeDtypeStruct((N, H), jnp.float32),
            jax.ShapeDtypeStruct((N, H), jnp.float32),
        ],
    )(accT, dinv16)


def _tc_mid(x1, h1, h2, h3, Wt0, Wt1, Wt2, Wt3, bt, Wc, Wr):
    N, H = x1.shape
    bN = 1000

    def body(x1_ref, h1_ref, h2_ref, h3_ref, w0, w1, w2, w3, b_ref, wc, wr, out_ref):
        s = (_dot(x1_ref[...], w0[...]) + _dot(h1_ref[...], w1[...])
             + _dot(h2_ref[...], w2[...]) + _dot(h3_ref[...], w3[...]) + b_ref[...])
        x2 = jnp.maximum(s, 0.0)
        out_ref[:, :H] = _dot(x2, wc[...])
        out_ref[:, H:] = _dot(x2, wr[...])

    node = pl.BlockSpec((bN, H), lambda i: (i, 0))
    wspec = pl.BlockSpec((H, H), lambda i: (0, 0))
    return pl.pallas_call(
        body,
        grid=(N // bN,),
        in_specs=[node, node, node, node, wspec, wspec, wspec, wspec,
                  pl.BlockSpec((1, H), lambda i: (0, 0)), wspec, wspec],
        out_specs=pl.BlockSpec((bN, 2 * H), lambda i: (i, 0)),
        out_shape=jax.ShapeDtypeStruct((N, 2 * H), jnp.float32),
    )(x1, h1, h2, h3, Wt0, Wt1, Wt2, Wt3, bt.reshape(1, -1), Wc, Wr)


def _tc_final(accB, deg16, Wb2, bb2):
    _, N, H = accB.shape
    OUT = Wb2.shape[1]
    bN = 1000

    def body(acc_ref, deg_ref, w_ref, b_ref, out_ref):
        S = acc_ref[0] + acc_ref[1]
        out_ref[...] = _dot(S, w_ref[...]) + deg_ref[:, 0:1] * b_ref[...]

    return pl.pallas_call(
        body,
        grid=(N // bN,),
        in_specs=[
            pl.BlockSpec((2, bN, H), lambda i: (0, i, 0)),
            pl.BlockSpec((bN, LANES), lambda i: (i, 0)),
            pl.BlockSpec((H, OUT), lambda i: (0, 0)),
            pl.BlockSpec((1, OUT), lambda i: (0, 0)),
        ],
        out_specs=pl.BlockSpec((bN, OUT), lambda i: (i, 0)),
        out_shape=jax.ShapeDtypeStruct((N, OUT), jnp.float32),
    )(accB, deg16, Wb2, bb2.reshape(1, -1))


def _pick_batch(epw, cap):
    for b in (128, 104, 80, 64, 56, 40, 32, 24, 16, 8):
        if b <= cap and epw % b == 0:
            return b
    return 8


@functools.lru_cache(maxsize=None)
def _sc_edge_phase(N, H, E):
    NW = NC * NS
    EPW = E // NW
    B = _pick_batch(EPW, 40)
    nb = EPW // B
    mesh = plsc.VectorSubcoreMesh(core_axis_name="c", subcore_axis_name="s")

    def body(PQ_hbm, C_hbm, row_hbm, col_hbm, zNH_hbm,
             acc_out, acc_sh, rowi, coli, bufc, bufr, bufC, tr, sem1, sem2):
        c = lax.axis_index("c")
        s = lax.axis_index("s")

        @pl.when(s == 0)
        def _():
            pltpu.sync_copy(zNH_hbm, acc_sh)

        plsc.subcore_barrier()
        w = c * NS + s

        def batch(b, carry):
            e0 = w * EPW + b * B
            pltpu.sync_copy(row_hbm.at[pl.ds(e0, B)], rowi)
            pltpu.sync_copy(col_hbm.at[pl.ds(e0, B)], coli)
            cpc = pltpu.async_copy(PQ_hbm.at[coli], bufc, sem1)
            cpr = pltpu.async_copy(PQ_hbm.at[rowi], bufr, sem2)
            pltpu.sync_copy(C_hbm.at[pl.ds(e0, B)], bufC)
            cpc.wait()
            cpr.wait()

            def pair(j, carry2):
                for k in range(H // LANES):
                    sl = pl.ds(k * LANES, LANES)
                    slq = pl.ds(H + k * LANES, LANES)
                    cc = bufC[j, sl]
                    tr[j, sl] = jnp.maximum(bufr[j, sl] + bufc[j, slq] + cc, 0.0)
                    bufC[j, sl] = jnp.maximum(bufc[j, sl] + bufr[j, slq] + cc, 0.0)
                return carry2
            lax.fori_loop(0, B, pair, 0)

            pltpu.sync_copy(bufC, acc_sh.at[coli], add=True)
            pltpu.sync_copy(tr, acc_sh.at[rowi], add=True)
            return carry

        lax.fori_loop(0, nb, batch, 0)
        plsc.subcore_barrier()

        for ci in range(NC):
            @pl.when(jnp.logical_and(s == 0, c == ci))
            def _(ci=ci):
                pltpu.sync_copy(acc_sh, acc_out.at[ci])

    return pl.kernel(
        body,
        out_type=jax.ShapeDtypeStruct((NC, N, H), jnp.float32),
        mesh=mesh,
        scratch_types=[
            pltpu.VMEM_SHARED((N, H), jnp.float32),
            pltpu.VMEM((B,), jnp.int32),
            pltpu.VMEM((B,), jnp.int32),
            pltpu.VMEM((B, 2 * H), jnp.float32),
            pltpu.VMEM((B, 2 * H), jnp.float32),
            pltpu.VMEM((B, H), jnp.float32),
            pltpu.VMEM((B, H), jnp.float32),
            pltpu.SemaphoreType.DMA,
            pltpu.SemaphoreType.DMA,
        ],
    )


@functools.lru_cache(maxsize=None)
def _sc_hop(N, H, E):
    NW = NC * NS
    EPW = E // NW
    B = _pick_batch(EPW, 80)
    nb = EPW // B
    mesh = plsc.VectorSubcoreMesh(core_axis_name="c", subcore_axis_name="s")

    def body(G_hbm, row_hbm, col_hbm, zNH_hbm, acc_out,
             acc_sh, rowi, coli, bufa, bufb, sem1, sem2):
        c = lax.axis_index("c")
        s = lax.axis_index("s")

        @pl.when(s == 0)
        def _():
            pltpu.sync_copy(zNH_hbm, acc_sh)

        plsc.subcore_barrier()
        w = c * NS + s

        def batch(b, carry):
            e0 = w * EPW + b * B
            pltpu.sync_copy(row_hbm.at[pl.ds(e0, B)], rowi)
            pltpu.sync_copy(col_hbm.at[pl.ds(e0, B)], coli)
            cpa = pltpu.async_copy(G_hbm.at[rowi], bufa, sem1)
            cpb = pltpu.async_copy(G_hbm.at[coli], bufb, sem2)
            cpa.wait()
            cpb.wait()
            pltpu.sync_copy(bufa, acc_sh.at[coli], add=True)
            pltpu.sync_copy(bufb, acc_sh.at[rowi], add=True)
            return carry

        lax.fori_loop(0, nb, batch, 0)
        plsc.subcore_barrier()

        for ci in range(NC):
            @pl.when(jnp.logical_and(s == 0, c == ci))
            def _(ci=ci):
                pltpu.sync_copy(acc_sh, acc_out.at[ci])

    return pl.kernel(
        body,
        out_type=jax.ShapeDtypeStruct((NC, N, H), jnp.float32),
        mesh=mesh,
        scratch_types=[
            pltpu.VMEM_SHARED((N, H), jnp.float32),
            pltpu.VMEM((B,), jnp.int32),
            pltpu.VMEM((B,), jnp.int32),
            pltpu.VMEM((B, H), jnp.float32),
            pltpu.VMEM((B, H), jnp.float32),
            pltpu.SemaphoreType.DMA,
            pltpu.SemaphoreType.DMA,
        ],
    )


@functools.lru_cache(maxsize=None)
def _sc_deg(N, H, E):
    NW = NC * NS
    EPW = E // NW
    B = _pick_batch(EPW, 80)
    nb = EPW // B
    mesh = plsc.VectorSubcoreMesh(core_axis_name="c", subcore_axis_name="s")

    def body(row_hbm, col_hbm, zNH_hbm, deg_out,
             acc_sh, rowi, coli, ones_v, sem1):
        c = lax.axis_index("c")
        s = lax.axis_index("s")

        @pl.when(s == 0)
        def _():
            pltpu.sync_copy(zNH_hbm, acc_sh)

        def fill_ones(j, carry):
            for k in range(H // LANES):
                ones_v[j, pl.ds(k * LANES, LANES)] = jnp.full((LANES,), 1.0, jnp.float32)
            return carry
        lax.fori_loop(0, B, fill_ones, 0)

        plsc.subcore_barrier()
        w = c * NS + s

        def batch(b, carry):
            e0 = w * EPW + b * B
            pltpu.sync_copy(row_hbm.at[pl.ds(e0, B)], rowi)
            pltpu.sync_copy(col_hbm.at[pl.ds(e0, B)], coli)
            pltpu.sync_copy(ones_v, acc_sh.at[coli], add=True)
            pltpu.sync_copy(ones_v, acc_sh.at[rowi], add=True)
            return carry

        lax.fori_loop(0, nb, batch, 0)
        plsc.subcore_barrier()

        for ci in range(NC):
            @pl.when(jnp.logical_and(s == 0, c == ci))
            def _(ci=ci):
                pltpu.sync_copy(acc_sh, deg_out.at[ci])

    return pl.kernel(
        body,
        out_type=jax.ShapeDtypeStruct((NC, N, H), jnp.float32),
        mesh=mesh,
        scratch_types=[
            pltpu.VMEM_SHARED((N, H), jnp.float32),
            pltpu.VMEM((B,), jnp.int32),
            pltpu.VMEM((B,), jnp.int32),
            pltpu.VMEM((B, H), jnp.float32),
            pltpu.SemaphoreType.DMA,
        ],
    )


def kernel(x, mask, edge_index, edge_attr,
           Wm1, bm1, Wm2, bm2, Wa1, ba1, Wa2, ba2,
           Wt, bt, Wb1, bb1, Wb2, bb2):
    N, F = x.shape
    E = edge_attr.shape[0]
    H = Wa1.shape[1]
    K = Wt.shape[0] // H - 1

    row = edge_index[0]
    col = edge_index[1]
    zNH = jnp.zeros((N, H), jnp.float32)

    PQ1 = _tc_prep(x, mask, Wm1, bm1, Wm2, bm2, Wa1[:F], Wa1[F:2 * F])
    C1 = _tc_edgepre(edge_attr, Wa1[2 * F:], ba1)
    degacc = _sc_deg(N, H, E)(row, col, zNH)
    accA = _sc_edge_phase(N, H, E)(PQ1, C1, row, col, zNH)
    x1, G, deg16, dinv16 = _tc_combine_a(accA, degacc, Wa2, ba2)

    hs = []
    for _ in range(K):
        accT = _sc_hop(N, H, E)(G, row, col, zNH)
        h, G = _tc_combine_t(accT, dinv16)
        hs.append(h)

    PQ2 = _tc_mid(x1, hs[0], hs[1], hs[2],
                  Wt[:H], Wt[H:2 * H], Wt[2 * H:3 * H], Wt[3 * H:],
                  bt, Wb1[:H], Wb1[H:2 * H])
    C2 = _tc_edgepre(edge_attr, Wb1[2 * H:], bb1)
    accB = _sc_edge_phase(N, H, E)(PQ2, C2, row, col, zNH)
    out = _tc_final(accB, deg16, Wb2, bb2)
    return out
